# baseline (speedup 1.0000x reference)
import functools

import jax
import jax.numpy as jnp
from jax import lax
from jax.experimental import pallas as pl
from jax.experimental.pallas import tpu as pltpu
from jax.experimental.pallas import tpu_sc as plsc

B = 16384
NCAT = 26
V = 100000
D = 16
NUM = 13
H1 = 128
H2 = 64
EPS = 1e-5

NC = 2
NS = 16
NW = NC * NS
R = B * NCAT
PER_W = R // NW
K = 3328
NCH = PER_W // K
DEPTH = 2


def _sc_gather_body(table, idxs, out, idx_v, *rest):
    rows = rest[:DEPTH]
    gsem = rest[DEPTH : 2 * DEPTH]
    osem = rest[2 * DEPTH : 3 * DEPTH]
    wid = lax.axis_index("s") * NC + lax.axis_index("c")
    base = wid * PER_W

    pltpu.sync_copy(idxs.at[pl.ds(base, PER_W)], idx_v)

    def fire(j, b):
        return pltpu.async_copy(
            table.at[idx_v.at[pl.ds(j * K, K)]], rows[b], gsem[b]
        )

    g = {}
    o = {}
    for j in range(min(DEPTH, NCH)):
        g[j] = fire(j, j)
    for j in range(NCH):
        b = j % DEPTH
        g[j].wait()
        o[j] = pltpu.async_copy(rows[b], out.at[pl.ds(base + j * K, K)], osem[b])
        n = j + DEPTH
        if n < NCH:
            o[j].wait()
            g[n] = fire(n, b)
    for j in range(max(NCH - DEPTH, 0), NCH):
        o[j].wait()


def _sc_gather(table, flat_idx):
    mesh = plsc.VectorSubcoreMesh(core_axis_name="c", subcore_axis_name="s")
    kern = pl.kernel(
        _sc_gather_body,
        out_type=jax.ShapeDtypeStruct((R, D), jnp.float32),
        mesh=mesh,
        scratch_types=[
            pltpu.VMEM((PER_W,), jnp.int32),
            *[pltpu.VMEM((K, D), jnp.float32) for _ in range(DEPTH)],
            *[pltpu.SemaphoreType.DMA for _ in range(2 * DEPTH)],
        ],
        compiler_params=pltpu.CompilerParams(use_tc_tiling_on_sc=False),
    )
    return kern(table, flat_idx)


VMAIN = (V // 128) * 128
VTAIL = V - VMAIN
RCH = 1408
NCHF = VMAIN // RCH
UPF = NCHF + 1
UNITS = NCAT * UPF
ULINES = RCH * D // 128
UWORDS = RCH * D
FWORDS = V * D
TWORDS = VTAIL * D


NQ = RCH // 128
MAIN = NCAT * NCHF
MPAIRS = ((MAIN + NW - 1) // NW + 1) // 2


def _sc_relayout_body(tbl, tail, out, slab_a, slab_b, stage_a, stage_b, tbuf,
                      isem_a, isem_b, osem_a, osem_b):
    wid = lax.axis_index("s") * NC + lax.axis_index("c")
    eiota = lax.iota(jnp.int32, 16)
    slabs = (slab_a, slab_b)
    stages = (stage_a, stage_b)
    isems = (isem_a, isem_b)
    osems = (osem_a, osem_b)

    def unit_ct(u):
        t = u % NCHF
        return u // NCHF, pl.multiple_of(t * RCH, 128)

    def fire_in(u, b):
        c, vb = unit_ct(u)
        for q in range(NQ):
            pltpu.async_copy(
                tbl.at[c, :, pl.ds(vb + q * 128, 128)],
                slabs[b].at[pl.ds(q * D, D)],
                isems[b],
            )

    def drain_in(u, b):
        c, vb = unit_ct(u)
        for q in range(NQ):
            pltpu.make_async_copy(
                tbl.at[c, :, pl.ds(vb + q * 128, 128)],
                slabs[b].at[pl.ds(q * D, D)],
                isems[b],
            ).wait()

    def compute(b):
        def line(rr):
            rowv = jnp.broadcast_to((rr // 16) * D, (16,)).astype(jnp.int32) + eiota
            vbase = jnp.broadcast_to((rr % 16) * 8, (16,)).astype(jnp.int32)
            gs = [
                plsc.load_gather(slabs[b], [rowv, vbase + k]) for k in range(8)
            ]
            for k in range(8):
                stages[b][pl.ds(rr * 128 + k * 16, 16)] = gs[k]

        plsc.parallel_loop(0, ULINES, unroll=8)(line)

    def drain_out(b):
        pltpu.make_async_copy(
            stages[b], out.at[pl.ds(0, UWORDS)], osems[b]
        ).wait()

    def fire_out(u, b):
        c, vb = unit_ct(u)
        pltpu.async_copy(
            stages[b], out.at[pl.ds(c * FWORDS + vb * D, UWORDS)], osems[b]
        )

    @pl.when(wid < MAIN)
    def _():
        fire_in(wid, 0)

    def pair(j, carry):
        u0 = wid + (2 * j) * NW
        u1 = wid + (2 * j + 1) * NW
        u2 = wid + (2 * j + 2) * NW

        @pl.when(u1 < MAIN)
        def _():
            fire_in(u1, 1)

        @pl.when(u0 < MAIN)
        def _():
            drain_in(u0, 0)

            @pl.when(j > 0)
            def _():
                drain_out(0)

            compute(0)
            fire_out(u0, 0)

        @pl.when(u2 < MAIN)
        def _():
            fire_in(u2, 0)

        @pl.when(u1 < MAIN)
        def _():
            drain_in(u1, 1)

            @pl.when(j > 0)
            def _():
                drain_out(1)

            compute(1)
            fire_out(u1, 1)

        return carry

    lax.fori_loop(0, MPAIRS, pair, 0)
    drain_out(0)
    drain_out(1)

    @pl.when(wid < NCAT)
    def _():
        pltpu.sync_copy(tail.at[pl.ds(wid * TWORDS, TWORDS)], tbuf)
        pltpu.sync_copy(tbuf, out.at[pl.ds(wid * FWORDS + VMAIN * D, TWORDS)])


def _sc_relayout(tbl_t, tail1d):
    mesh = plsc.VectorSubcoreMesh(core_axis_name="c", subcore_axis_name="s")
    kern = pl.kernel(
        _sc_relayout_body,
        out_type=jax.ShapeDtypeStruct((NCAT * V * D,), jnp.float32),
        mesh=mesh,
        scratch_types=[
            pltpu.VMEM((NQ * D, 128), jnp.float32),
            pltpu.VMEM((NQ * D, 128), jnp.float32),
            pltpu.VMEM((UWORDS,), jnp.float32),
            pltpu.VMEM((UWORDS,), jnp.float32),
            pltpu.VMEM((TWORDS,), jnp.float32),
            pltpu.SemaphoreType.DMA,
            pltpu.SemaphoreType.DMA,
            pltpu.SemaphoreType.DMA,
            pltpu.SemaphoreType.DMA,
        ],
        compiler_params=pltpu.CompilerParams(
            use_tc_tiling_on_sc=True, needs_layout_passes=False
        ),
    )
    return kern(tbl_t, tail1d)


BS = 2048
NB = B // BS


def _mlp_body(xcat, num, w1c, w1n, b1, g1, be1, w2, b2, g2, be2, w3, b3,
              out, h1_s, h2_s, acc1, acc2, m1, m2):
    p = pl.program_id(0)
    i = pl.program_id(1)
    off = pl.multiple_of(i * BS, BS)

    @pl.when(p == 0)
    def _phase0():
        @pl.when(i == 0)
        def _():
            acc1[...] = jnp.zeros_like(acc1)

        h = jnp.dot(xcat[...], w1c[...], preferred_element_type=jnp.float32)
        h = h + jnp.dot(num[...], w1n[...], preferred_element_type=jnp.float32)
        h = h + b1[...]
        h1_s[pl.ds(off, BS), :] = h
        acc1[0:1, :] += jnp.sum(h, axis=0, keepdims=True)
        acc1[1:2, :] += jnp.sum(h * h, axis=0, keepdims=True)

        @pl.when(i == NB - 1)
        def _():
            mean = acc1[0:1, :] * (1.0 / B)
            var = acc1[1:2, :] * (1.0 / B) - mean * mean
            m1[0:1, :] = mean
            m1[1:2, :] = lax.rsqrt(var + EPS)

    @pl.when(p == 1)
    def _phase1():
        @pl.when(i == 0)
        def _():
            acc2[...] = jnp.zeros_like(acc2)

        h = h1_s[pl.ds(off, BS), :]
        h = (h - m1[0:1, :]) * (m1[1:2, :] * g1[...]) + be1[...]
        h = jnp.maximum(h, 0.0)
        h2 = jnp.dot(h, w2[...], preferred_element_type=jnp.float32) + b2[...]
        h2_s[pl.ds(off, BS), :] = h2
        acc2[0:1, :] += jnp.sum(h2, axis=0, keepdims=True)
        acc2[1:2, :] += jnp.sum(h2 * h2, axis=0, keepdims=True)

        @pl.when(i == NB - 1)
        def _():
            mean = acc2[0:1, :] * (1.0 / B)
            var = acc2[1:2, :] * (1.0 / B) - mean * mean
            m2[0:1, :] = mean
            m2[1:2, :] = lax.rsqrt(var + EPS)

    @pl.when(p == 2)
    def _phase2():
        h2 = h2_s[pl.ds(off, BS), :]
        h2 = (h2 - m2[0:1, :]) * (m2[1:2, :] * g2[...]) + be2[...]
        h2 = jnp.maximum(h2, 0.0)
        o = jnp.sum(h2 * w3[...], axis=1, keepdims=True) + b3[...]
        out[...] = o


def _mlp(xcat_2d, num, w1c_t, w1n_t, b1, g1, be1, w2_t, b2, g2, be2, w3, b3):
    grid = (3, NB)

    def xmap(p, i):
        return (jnp.where(p == 0, i, 0), 0)

    const = lambda p, i: (0, 0)
    return pl.pallas_call(
        _mlp_body,
        grid=grid,
        in_specs=[
            pl.BlockSpec((BS, NCAT * D), xmap),
            pl.BlockSpec((BS, NUM), xmap),
            pl.BlockSpec((NCAT * D, H1), const),
            pl.BlockSpec((NUM, H1), const),
            pl.BlockSpec((1, H1), const),
            pl.BlockSpec((1, H1), const),
            pl.BlockSpec((1, H1), const),
            pl.BlockSpec((H1, H2), const),
            pl.BlockSpec((1, H2), const),
            pl.BlockSpec((1, H2), const),
            pl.BlockSpec((1, H2), const),
            pl.BlockSpec((1, H2), const),
            pl.BlockSpec((1, 1), const),
        ],
        out_specs=pl.BlockSpec((BS, 1), lambda p, i: (i, 0)),
        out_shape=jax.ShapeDtypeStruct((B, 1), jnp.float32),
        scratch_shapes=[
            pltpu.VMEM((B, H1), jnp.float32),
            pltpu.VMEM((B, H2), jnp.float32),
            pltpu.VMEM((2, H1), jnp.float32),
            pltpu.VMEM((2, H2), jnp.float32),
            pltpu.VMEM((2, H1), jnp.float32),
            pltpu.VMEM((2, H2), jnp.float32),
        ],
        compiler_params=pltpu.CompilerParams(
            dimension_semantics=("arbitrary", "arbitrary"),
        ),
    )(xcat_2d, num, w1c_t, w1n_t, b1, g1, be1, w2_t, b2, g2, be2, w3, b3)


def kernel(cat, num, emb_tables, W1, b1, g1, beta1, W2, b2, g2, beta2, W3, b3):
    flat_idx = (cat + (jnp.arange(NCAT, dtype=jnp.int32) * V)[None, :]).reshape(R)
    tbl_t = jnp.transpose(emb_tables, (0, 2, 1))
    tail1d = emb_tables[:, VMAIN:, :].reshape(NCAT * TWORDS)
    table = _sc_relayout(tbl_t, tail1d).reshape(NCAT * V, D)

    gathered = _sc_gather(table, flat_idx)
    xcat = gathered.reshape(B, NCAT * D)

    w1t = W1.T
    out = _mlp(
        xcat, num,
        w1t[: NCAT * D, :], w1t[NCAT * D :, :],
        b1.reshape(1, H1), g1.reshape(1, H1), beta1.reshape(1, H1),
        W2.T, b2.reshape(1, H2), g2.reshape(1, H2), beta2.reshape(1, H2),
        W3,
        b3.reshape(1, 1),
    )
    return out


# speedup vs baseline: 1.2724x; 1.2724x over previous
import functools

import jax
import jax.numpy as jnp
from jax import lax
from jax.experimental import pallas as pl
from jax.experimental.pallas import tpu as pltpu
from jax.experimental.pallas import tpu_sc as plsc

B = 16384
NCAT = 26
V = 100000
D = 16
NUM = 13
H1 = 128
H2 = 64
EPS = 1e-5

NC = 2
NS = 16
NW = NC * NS
R = B * NCAT
PER_W = R // NW
K = 3328
NCH = PER_W // K
DEPTH = 2


def _sc_gather_body(table, idxs, out, idx_v, *rest):
    rows = rest[:DEPTH]
    gsem = rest[DEPTH : 2 * DEPTH]
    osem = rest[2 * DEPTH : 3 * DEPTH]
    wid = lax.axis_index("s") * NC + lax.axis_index("c")
    base = wid * PER_W

    pltpu.sync_copy(idxs.at[pl.ds(base, PER_W)], idx_v)

    def fire(j, b):
        return pltpu.async_copy(
            table.at[idx_v.at[pl.ds(j * K, K)]], rows[b], gsem[b]
        )

    g = {}
    o = {}
    for j in range(min(DEPTH, NCH)):
        g[j] = fire(j, j)
    for j in range(NCH):
        b = j % DEPTH
        g[j].wait()
        o[j] = pltpu.async_copy(rows[b], out.at[pl.ds(base + j * K, K)], osem[b])
        n = j + DEPTH
        if n < NCH:
            o[j].wait()
            g[n] = fire(n, b)
    for j in range(max(NCH - DEPTH, 0), NCH):
        o[j].wait()


def _sc_gather(table, flat_idx):
    mesh = plsc.VectorSubcoreMesh(core_axis_name="c", subcore_axis_name="s")
    kern = pl.kernel(
        _sc_gather_body,
        out_type=jax.ShapeDtypeStruct((R, D), jnp.float32),
        mesh=mesh,
        scratch_types=[
            pltpu.VMEM((PER_W,), jnp.int32),
            *[pltpu.VMEM((K, D), jnp.float32) for _ in range(DEPTH)],
            *[pltpu.SemaphoreType.DMA for _ in range(2 * DEPTH)],
        ],
        compiler_params=pltpu.CompilerParams(use_tc_tiling_on_sc=False),
    )
    return kern(table, flat_idx)


VMAIN = (V // 128) * 128
VTAIL = V - VMAIN
RCH = 1408
NCHF = VMAIN // RCH
UPF = NCHF + 1
UNITS = NCAT * UPF
ULINES = RCH * D // 128
UWORDS = RCH * D
FWORDS = V * D
TWORDS = VTAIL * D


NQ = RCH // 128
MAIN = NCAT * NCHF
MPAIRS = ((MAIN + NW - 1) // NW + 1) // 2


def _sc_relayout_body(tbl, tail, out, slab_a, slab_b, stage_a, stage_b, tbuf,
                      isem_a, isem_b, osem_a, osem_b):
    wid = lax.axis_index("s") * NC + lax.axis_index("c")
    eiota = lax.iota(jnp.int32, 16)
    slabs = (slab_a, slab_b)
    stages = (stage_a, stage_b)
    isems = (isem_a, isem_b)
    osems = (osem_a, osem_b)

    def unit_ct(u):
        t = u % NCHF
        return u // NCHF, pl.multiple_of(t * RCH, 128)

    def fire_in(u, b):
        c, vb = unit_ct(u)
        for q in range(NQ):
            pltpu.async_copy(
                tbl.at[c, :, pl.ds(vb + q * 128, 128)], slabs[b].at[q], isems[b]
            )

    def drain_in(u, b):
        c, vb = unit_ct(u)
        for q in range(NQ):
            pltpu.make_async_copy(
                tbl.at[c, :, pl.ds(vb + q * 128, 128)], slabs[b].at[q], isems[b]
            ).wait()

    def compute(b):
        for q in range(NQ):

            def line(r2, q=q):
                vbase = jnp.broadcast_to(r2 * 8, (16,)).astype(jnp.int32)
                gs = [
                    plsc.load_gather(slabs[b].at[q], [eiota, vbase + k])
                    for k in range(8)
                ]
                for k in range(8):
                    stages[b][pl.ds(q * 2048 + r2 * 128 + k * 16, 16)] = gs[k]

            plsc.parallel_loop(0, 16, unroll=8)(line)

    def drain_out(b):
        pltpu.make_async_copy(
            stages[b], out.at[pl.ds(0, UWORDS)], osems[b]
        ).wait()

    def fire_out(u, b):
        c, vb = unit_ct(u)
        pltpu.async_copy(
            stages[b], out.at[pl.ds(c * FWORDS + vb * D, UWORDS)], osems[b]
        )

    @pl.when(wid < MAIN)
    def _():
        fire_in(wid, 0)

    def pair(j, carry):
        u0 = wid + (2 * j) * NW
        u1 = wid + (2 * j + 1) * NW
        u2 = wid + (2 * j + 2) * NW

        @pl.when(u1 < MAIN)
        def _():
            fire_in(u1, 1)

        @pl.when(u0 < MAIN)
        def _():
            drain_in(u0, 0)

            @pl.when(j > 0)
            def _():
                drain_out(0)

            compute(0)
            fire_out(u0, 0)

        @pl.when(u2 < MAIN)
        def _():
            fire_in(u2, 0)

        @pl.when(u1 < MAIN)
        def _():
            drain_in(u1, 1)

            @pl.when(j > 0)
            def _():
                drain_out(1)

            compute(1)
            fire_out(u1, 1)

        return carry

    lax.fori_loop(0, MPAIRS, pair, 0)
    drain_out(0)
    drain_out(1)

    @pl.when(wid < NCAT)
    def _():
        pltpu.sync_copy(tail.at[pl.ds(wid * TWORDS, TWORDS)], tbuf)
        pltpu.sync_copy(tbuf, out.at[pl.ds(wid * FWORDS + VMAIN * D, TWORDS)])


def _sc_relayout(tbl_t, tail1d):
    mesh = plsc.VectorSubcoreMesh(core_axis_name="c", subcore_axis_name="s")
    kern = pl.kernel(
        _sc_relayout_body,
        out_type=jax.ShapeDtypeStruct((NCAT * V * D,), jnp.float32),
        mesh=mesh,
        scratch_types=[
            pltpu.VMEM((NQ, D, 128), jnp.float32),
            pltpu.VMEM((NQ, D, 128), jnp.float32),
            pltpu.VMEM((UWORDS,), jnp.float32),
            pltpu.VMEM((UWORDS,), jnp.float32),
            pltpu.VMEM((TWORDS,), jnp.float32),
            pltpu.SemaphoreType.DMA,
            pltpu.SemaphoreType.DMA,
            pltpu.SemaphoreType.DMA,
            pltpu.SemaphoreType.DMA,
        ],
        compiler_params=pltpu.CompilerParams(
            use_tc_tiling_on_sc=True, needs_layout_passes=False
        ),
    )
    return kern(tbl_t, tail1d)


BS = 2048
NB = B // BS


def _mlp_body(xcat, num, w1c, w1n, b1, g1, be1, w2, b2, g2, be2, w3, b3,
              out, h1_s, h2_s, acc1, acc2, m1, m2):
    p = pl.program_id(0)
    i = pl.program_id(1)
    off = pl.multiple_of(i * BS, BS)

    @pl.when(p == 0)
    def _phase0():
        @pl.when(i == 0)
        def _():
            acc1[...] = jnp.zeros_like(acc1)

        h = jnp.dot(xcat[...], w1c[...], preferred_element_type=jnp.float32)
        h = h + jnp.dot(num[...], w1n[...], preferred_element_type=jnp.float32)
        h = h + b1[...]
        h1_s[pl.ds(off, BS), :] = h
        acc1[0:1, :] += jnp.sum(h, axis=0, keepdims=True)
        acc1[1:2, :] += jnp.sum(h * h, axis=0, keepdims=True)

        @pl.when(i == NB - 1)
        def _():
            mean = acc1[0:1, :] * (1.0 / B)
            var = acc1[1:2, :] * (1.0 / B) - mean * mean
            m1[0:1, :] = mean
            m1[1:2, :] = lax.rsqrt(var + EPS)

    @pl.when(p == 1)
    def _phase1():
        @pl.when(i == 0)
        def _():
            acc2[...] = jnp.zeros_like(acc2)

        h = h1_s[pl.ds(off, BS), :]
        h = (h - m1[0:1, :]) * (m1[1:2, :] * g1[...]) + be1[...]
        h = jnp.maximum(h, 0.0)
        h2 = jnp.dot(h, w2[...], preferred_element_type=jnp.float32) + b2[...]
        h2_s[pl.ds(off, BS), :] = h2
        acc2[0:1, :] += jnp.sum(h2, axis=0, keepdims=True)
        acc2[1:2, :] += jnp.sum(h2 * h2, axis=0, keepdims=True)

        @pl.when(i == NB - 1)
        def _():
            mean = acc2[0:1, :] * (1.0 / B)
            var = acc2[1:2, :] * (1.0 / B) - mean * mean
            m2[0:1, :] = mean
            m2[1:2, :] = lax.rsqrt(var + EPS)

    @pl.when(p == 2)
    def _phase2():
        h2 = h2_s[pl.ds(off, BS), :]
        h2 = (h2 - m2[0:1, :]) * (m2[1:2, :] * g2[...]) + be2[...]
        h2 = jnp.maximum(h2, 0.0)
        o = jnp.sum(h2 * w3[...], axis=1, keepdims=True) + b3[...]
        out[...] = o


def _mlp(xcat_2d, num, w1c_t, w1n_t, b1, g1, be1, w2_t, b2, g2, be2, w3, b3):
    grid = (3, NB)

    def xmap(p, i):
        return (jnp.where(p == 0, i, 0), 0)

    const = lambda p, i: (0, 0)
    return pl.pallas_call(
        _mlp_body,
        grid=grid,
        in_specs=[
            pl.BlockSpec((BS, NCAT * D), xmap),
            pl.BlockSpec((BS, NUM), xmap),
            pl.BlockSpec((NCAT * D, H1), const),
            pl.BlockSpec((NUM, H1), const),
            pl.BlockSpec((1, H1), const),
            pl.BlockSpec((1, H1), const),
            pl.BlockSpec((1, H1), const),
            pl.BlockSpec((H1, H2), const),
            pl.BlockSpec((1, H2), const),
            pl.BlockSpec((1, H2), const),
            pl.BlockSpec((1, H2), const),
            pl.BlockSpec((1, H2), const),
            pl.BlockSpec((1, 1), const),
        ],
        out_specs=pl.BlockSpec((BS, 1), lambda p, i: (i, 0)),
        out_shape=jax.ShapeDtypeStruct((B, 1), jnp.float32),
        scratch_shapes=[
            pltpu.VMEM((B, H1), jnp.float32),
            pltpu.VMEM((B, H2), jnp.float32),
            pltpu.VMEM((2, H1), jnp.float32),
            pltpu.VMEM((2, H2), jnp.float32),
            pltpu.VMEM((2, H1), jnp.float32),
            pltpu.VMEM((2, H2), jnp.float32),
        ],
        compiler_params=pltpu.CompilerParams(
            dimension_semantics=("arbitrary", "arbitrary"),
        ),
    )(xcat_2d, num, w1c_t, w1n_t, b1, g1, be1, w2_t, b2, g2, be2, w3, b3)


def kernel(cat, num, emb_tables, W1, b1, g1, beta1, W2, b2, g2, beta2, W3, b3):
    flat_idx = (cat + (jnp.arange(NCAT, dtype=jnp.int32) * V)[None, :]).reshape(R)
    tbl_t = jnp.transpose(emb_tables, (0, 2, 1))
    tail1d = emb_tables[:, VMAIN:, :].reshape(NCAT * TWORDS)
    table = _sc_relayout(tbl_t, tail1d).reshape(NCAT * V, D)

    gathered = _sc_gather(table, flat_idx)
    xcat = gathered.reshape(B, NCAT * D)

    w1t = W1.T
    out = _mlp(
        xcat, num,
        w1t[: NCAT * D, :], w1t[NCAT * D :, :],
        b1.reshape(1, H1), g1.reshape(1, H1), beta1.reshape(1, H1),
        W2.T, b2.reshape(1, H2), g2.reshape(1, H2), beta2.reshape(1, H2),
        W3,
        b3.reshape(1, 1),
    )
    return out


# speedup vs baseline: 1.2830x; 1.0084x over previous
import functools

import jax
import jax.numpy as jnp
from jax import lax
from jax.experimental import pallas as pl
from jax.experimental.pallas import tpu as pltpu
from jax.experimental.pallas import tpu_sc as plsc

B = 16384
NCAT = 26
V = 100000
D = 16
NUM = 13
H1 = 128
H2 = 64
EPS = 1e-5

NC = 2
NS = 16
NW = NC * NS
R = B * NCAT
PER_W = R // NW
K = 3328
NCH = PER_W // K
DEPTH = 2


def _sc_gather_body(table, idxs, out, idx_v, *rest):
    rows = rest[:DEPTH]
    gsem = rest[DEPTH : 2 * DEPTH]
    osem = rest[2 * DEPTH : 3 * DEPTH]
    wid = lax.axis_index("s") * NC + lax.axis_index("c")
    base = wid * PER_W

    pltpu.sync_copy(idxs.at[pl.ds(base, PER_W)], idx_v)

    eiota = lax.iota(jnp.int32, 16)

    def fixidx(j):
        phase = (j * 16) % NCAT
        offs = ((eiota + phase) % NCAT) * V
        idx_v[pl.ds(j * 16, 16)] = idx_v[pl.ds(j * 16, 16)] + offs

    plsc.parallel_loop(0, PER_W // 16, unroll=4)(fixidx)

    def fire(j, b):
        return pltpu.async_copy(
            table.at[idx_v.at[pl.ds(j * K, K)]], rows[b], gsem[b]
        )

    g = {}
    o = {}
    for j in range(min(DEPTH, NCH)):
        g[j] = fire(j, j)
    for j in range(NCH):
        b = j % DEPTH
        g[j].wait()
        o[j] = pltpu.async_copy(rows[b], out.at[pl.ds(base + j * K, K)], osem[b])
        n = j + DEPTH
        if n < NCH:
            o[j].wait()
            g[n] = fire(n, b)
    for j in range(max(NCH - DEPTH, 0), NCH):
        o[j].wait()


def _sc_gather(table, flat_idx):
    mesh = plsc.VectorSubcoreMesh(core_axis_name="c", subcore_axis_name="s")
    kern = pl.kernel(
        _sc_gather_body,
        out_type=jax.ShapeDtypeStruct((R, D), jnp.float32),
        mesh=mesh,
        scratch_types=[
            pltpu.VMEM((PER_W,), jnp.int32),
            *[pltpu.VMEM((K, D), jnp.float32) for _ in range(DEPTH)],
            *[pltpu.SemaphoreType.DMA for _ in range(2 * DEPTH)],
        ],
        compiler_params=pltpu.CompilerParams(use_tc_tiling_on_sc=False),
    )
    return kern(table, flat_idx)


VMAIN = (V // 128) * 128
VTAIL = V - VMAIN
RCH = 1408
NCHF = VMAIN // RCH
UPF = NCHF + 1
UNITS = NCAT * UPF
ULINES = RCH * D // 128
UWORDS = RCH * D
FWORDS = V * D
TWORDS = VTAIL * D


NQ = RCH // 128
MAIN = NCAT * NCHF
MPAIRS = ((MAIN + NW - 1) // NW + 1) // 2


def _sc_relayout_body(tbl, tail, out, slab_a, slab_b, stage_a, stage_b, tbuf,
                      isem_a, isem_b, osem_a, osem_b):
    wid = lax.axis_index("s") * NC + lax.axis_index("c")
    eiota = lax.iota(jnp.int32, 16)
    slabs = (slab_a, slab_b)
    stages = (stage_a, stage_b)
    isems = (isem_a, isem_b)
    osems = (osem_a, osem_b)

    def unit_ct(u):
        t = u % NCHF
        return u // NCHF, pl.multiple_of(t * RCH, 128)

    def fire_in(u, b):
        c, vb = unit_ct(u)
        for q in range(NQ):
            pltpu.async_copy(
                tbl.at[c, :, pl.ds(vb + q * 128, 128)], slabs[b].at[q], isems[b]
            )

    def drain_in(u, b):
        c, vb = unit_ct(u)
        for q in range(NQ):
            pltpu.make_async_copy(
                tbl.at[c, :, pl.ds(vb + q * 128, 128)], slabs[b].at[q], isems[b]
            ).wait()

    def compute(b):
        for q in range(NQ):

            def line(r2, q=q):
                vbase = jnp.broadcast_to(r2 * 8, (16,)).astype(jnp.int32)
                gs = [
                    plsc.load_gather(slabs[b].at[q], [eiota, vbase + k])
                    for k in range(8)
                ]
                for k in range(8):
                    stages[b][pl.ds(q * 2048 + r2 * 128 + k * 16, 16)] = gs[k]

            plsc.parallel_loop(0, 16, unroll=4)(line)

    def drain_out(b):
        pltpu.make_async_copy(
            stages[b], out.at[pl.ds(0, UWORDS)], osems[b]
        ).wait()

    def fire_out(u, b):
        c, vb = unit_ct(u)
        pltpu.async_copy(
            stages[b], out.at[pl.ds(c * FWORDS + vb * D, UWORDS)], osems[b]
        )

    @pl.when(wid < MAIN)
    def _():
        fire_in(wid, 0)

    def pair(j, carry):
        u0 = wid + (2 * j) * NW
        u1 = wid + (2 * j + 1) * NW
        u2 = wid + (2 * j + 2) * NW

        @pl.when(u1 < MAIN)
        def _():
            fire_in(u1, 1)

        @pl.when(u0 < MAIN)
        def _():
            drain_in(u0, 0)

            @pl.when(j > 0)
            def _():
                drain_out(0)

            compute(0)
            fire_out(u0, 0)

        @pl.when(u2 < MAIN)
        def _():
            fire_in(u2, 0)

        @pl.when(u1 < MAIN)
        def _():
            drain_in(u1, 1)

            @pl.when(j > 0)
            def _():
                drain_out(1)

            compute(1)
            fire_out(u1, 1)

        return carry

    lax.fori_loop(0, MPAIRS, pair, 0)
    drain_out(0)
    drain_out(1)

    @pl.when(wid < NCAT)
    def _():
        pltpu.sync_copy(tail.at[pl.ds(wid * TWORDS, TWORDS)], tbuf)
        pltpu.sync_copy(tbuf, out.at[pl.ds(wid * FWORDS + VMAIN * D, TWORDS)])


def _sc_relayout(tbl_t, tail1d):
    mesh = plsc.VectorSubcoreMesh(core_axis_name="c", subcore_axis_name="s")
    kern = pl.kernel(
        _sc_relayout_body,
        out_type=jax.ShapeDtypeStruct((NCAT * V * D,), jnp.float32),
        mesh=mesh,
        scratch_types=[
            pltpu.VMEM((NQ, D, 128), jnp.float32),
            pltpu.VMEM((NQ, D, 128), jnp.float32),
            pltpu.VMEM((UWORDS,), jnp.float32),
            pltpu.VMEM((UWORDS,), jnp.float32),
            pltpu.VMEM((TWORDS,), jnp.float32),
            pltpu.SemaphoreType.DMA,
            pltpu.SemaphoreType.DMA,
            pltpu.SemaphoreType.DMA,
            pltpu.SemaphoreType.DMA,
        ],
        compiler_params=pltpu.CompilerParams(
            use_tc_tiling_on_sc=True, needs_layout_passes=False
        ),
    )
    return kern(tbl_t, tail1d)


BS = 2048
NB = B // BS


def _mlp_body(xcat, num, w1c, w1n, b1, g1, be1, w2, b2, g2, be2, w3, b3,
              out, h1_s, h2_s, acc1, acc2, m1, m2):
    p = pl.program_id(0)
    i = pl.program_id(1)
    off = pl.multiple_of(i * BS, BS)

    @pl.when(p == 0)
    def _phase0():
        @pl.when(i == 0)
        def _():
            acc1[...] = jnp.zeros_like(acc1)

        h = jnp.dot(xcat[...], w1c[...], preferred_element_type=jnp.float32)
        h = h + jnp.dot(num[...], w1n[...], preferred_element_type=jnp.float32)
        h = h + b1[...]
        h1_s[pl.ds(off, BS), :] = h
        acc1[0:1, :] += jnp.sum(h, axis=0, keepdims=True)
        acc1[1:2, :] += jnp.sum(h * h, axis=0, keepdims=True)

        @pl.when(i == NB - 1)
        def _():
            mean = acc1[0:1, :] * (1.0 / B)
            var = acc1[1:2, :] * (1.0 / B) - mean * mean
            m1[0:1, :] = mean
            m1[1:2, :] = lax.rsqrt(var + EPS)

    @pl.when(p == 1)
    def _phase1():
        @pl.when(i == 0)
        def _():
            acc2[...] = jnp.zeros_like(acc2)

        h = h1_s[pl.ds(off, BS), :]
        h = (h - m1[0:1, :]) * (m1[1:2, :] * g1[...]) + be1[...]
        h = jnp.maximum(h, 0.0)
        h2 = jnp.dot(h, w2[...], preferred_element_type=jnp.float32) + b2[...]
        h2_s[pl.ds(off, BS), :] = h2
        acc2[0:1, :] += jnp.sum(h2, axis=0, keepdims=True)
        acc2[1:2, :] += jnp.sum(h2 * h2, axis=0, keepdims=True)

        @pl.when(i == NB - 1)
        def _():
            mean = acc2[0:1, :] * (1.0 / B)
            var = acc2[1:2, :] * (1.0 / B) - mean * mean
            m2[0:1, :] = mean
            m2[1:2, :] = lax.rsqrt(var + EPS)

    @pl.when(p == 2)
    def _phase2():
        h2 = h2_s[pl.ds(off, BS), :]
        h2 = (h2 - m2[0:1, :]) * (m2[1:2, :] * g2[...]) + be2[...]
        h2 = jnp.maximum(h2, 0.0)
        o = jnp.sum(h2 * w3[...], axis=1, keepdims=True) + b3[...]
        out[...] = o


def _mlp(xcat_2d, num, w1c_t, w1n_t, b1, g1, be1, w2_t, b2, g2, be2, w3, b3):
    grid = (3, NB)

    def xmap(p, i):
        return (jnp.where(p == 0, i, 0), 0)

    const = lambda p, i: (0, 0)
    return pl.pallas_call(
        _mlp_body,
        grid=grid,
        in_specs=[
            pl.BlockSpec((BS, NCAT * D), xmap),
            pl.BlockSpec((BS, NUM), xmap),
            pl.BlockSpec((NCAT * D, H1), const),
            pl.BlockSpec((NUM, H1), const),
            pl.BlockSpec((1, H1), const),
            pl.BlockSpec((1, H1), const),
            pl.BlockSpec((1, H1), const),
            pl.BlockSpec((H1, H2), const),
            pl.BlockSpec((1, H2), const),
            pl.BlockSpec((1, H2), const),
            pl.BlockSpec((1, H2), const),
            pl.BlockSpec((1, H2), const),
            pl.BlockSpec((1, 1), const),
        ],
        out_specs=pl.BlockSpec((BS, 1), lambda p, i: (i, 0)),
        out_shape=jax.ShapeDtypeStruct((B, 1), jnp.float32),
        scratch_shapes=[
            pltpu.VMEM((B, H1), jnp.float32),
            pltpu.VMEM((B, H2), jnp.float32),
            pltpu.VMEM((2, H1), jnp.float32),
            pltpu.VMEM((2, H2), jnp.float32),
            pltpu.VMEM((2, H1), jnp.float32),
            pltpu.VMEM((2, H2), jnp.float32),
        ],
        compiler_params=pltpu.CompilerParams(
            dimension_semantics=("arbitrary", "arbitrary"),
        ),
    )(xcat_2d, num, w1c_t, w1n_t, b1, g1, be1, w2_t, b2, g2, be2, w3, b3)


def kernel(cat, num, emb_tables, W1, b1, g1, beta1, W2, b2, g2, beta2, W3, b3):
    flat_idx = cat.reshape(R)
    tbl_t = jnp.transpose(emb_tables, (0, 2, 1))
    tail1d = emb_tables[:, VMAIN:, :].reshape(NCAT * TWORDS)
    table = _sc_relayout(tbl_t, tail1d).reshape(NCAT * V, D)

    gathered = _sc_gather(table, flat_idx)
    xcat = gathered.reshape(B, NCAT * D)

    w1t = W1.T
    out = _mlp(
        xcat, num,
        w1t[: NCAT * D, :], w1t[NCAT * D :, :],
        b1.reshape(1, H1), g1.reshape(1, H1), beta1.reshape(1, H1),
        W2.T, b2.reshape(1, H2), g2.reshape(1, H2), beta2.reshape(1, H2),
        W3,
        b3.reshape(1, 1),
    )
    return out


# speedup vs baseline: 1.3038x; 1.0162x over previous
import functools

import jax
import jax.numpy as jnp
from jax import lax
from jax.experimental import pallas as pl
from jax.experimental.pallas import tpu as pltpu
from jax.experimental.pallas import tpu_sc as plsc

B = 16384
NCAT = 26
V = 100000
D = 16
NUM = 13
H1 = 128
H2 = 64
EPS = 1e-5

NC = 2
NS = 16
NW = NC * NS
R = B * NCAT
PER_W = R // NW
K = 3328
NCH = PER_W // K
DEPTH = 2


def _sc_gather_body(table, idxs, out, idx_v, *rest):
    rows = rest[:DEPTH]
    gsem = rest[DEPTH : 2 * DEPTH]
    osem = rest[2 * DEPTH : 3 * DEPTH]
    wid = lax.axis_index("s") * NC + lax.axis_index("c")
    base = wid * PER_W

    pltpu.sync_copy(idxs.at[pl.ds(base, PER_W)], idx_v)

    eiota = lax.iota(jnp.int32, 16)

    def fixidx(j):
        phase = (j * 16) % NCAT
        offs = ((eiota + phase) % NCAT) * V
        idx_v[pl.ds(j * 16, 16)] = idx_v[pl.ds(j * 16, 16)] + offs

    plsc.parallel_loop(0, PER_W // 16, unroll=4)(fixidx)

    def fire(j, b):
        return pltpu.async_copy(
            table.at[idx_v.at[pl.ds(j * K, K)]], rows[b], gsem[b]
        )

    g = {}
    o = {}
    for j in range(min(DEPTH, NCH)):
        g[j] = fire(j, j)
    for j in range(NCH):
        b = j % DEPTH
        g[j].wait()
        o[j] = pltpu.async_copy(rows[b], out.at[pl.ds(base + j * K, K)], osem[b])
        n = j + DEPTH
        if n < NCH:
            o[j].wait()
            g[n] = fire(n, b)
    for j in range(max(NCH - DEPTH, 0), NCH):
        o[j].wait()


def _sc_gather(table, flat_idx):
    mesh = plsc.VectorSubcoreMesh(core_axis_name="c", subcore_axis_name="s")
    kern = pl.kernel(
        _sc_gather_body,
        out_type=jax.ShapeDtypeStruct((R, D), jnp.float32),
        mesh=mesh,
        scratch_types=[
            pltpu.VMEM((PER_W,), jnp.int32),
            *[pltpu.VMEM((K, D), jnp.float32) for _ in range(DEPTH)],
            *[pltpu.SemaphoreType.DMA for _ in range(2 * DEPTH)],
        ],
        compiler_params=pltpu.CompilerParams(use_tc_tiling_on_sc=False),
    )
    return kern(table, flat_idx)


VMAIN = (V // 128) * 128
VTAIL = V - VMAIN
RCH = 1408
NCHF = VMAIN // RCH
UPF = NCHF + 1
UNITS = NCAT * UPF
ULINES = RCH * D // 128
UWORDS = RCH * D
FWORDS = V * D
TWORDS = VTAIL * D


NQ = RCH // 128
MAIN = NCAT * NCHF
MPAIRS = ((MAIN + NW - 1) // NW + 1) // 2


def _sc_relayout_body(tbl, tail, out, slab_a, slab_b, stage_a, stage_b, tbuf,
                      isem_a, isem_b, osem_a, osem_b):
    wid = lax.axis_index("s") * NC + lax.axis_index("c")
    eiota = lax.iota(jnp.int32, 16)
    slabs = (slab_a, slab_b)
    stages = (stage_a, stage_b)
    isems = (isem_a, isem_b)
    osems = (osem_a, osem_b)

    def unit_ct(u):
        t = u % NCHF
        return u // NCHF, pl.multiple_of(t * RCH, 128)

    def fire_in(u, b):
        c, vb = unit_ct(u)
        pltpu.async_copy(tbl.at[c, :, pl.ds(vb, RCH)], slabs[b], isems[b])

    def drain_in(u, b):
        c, vb = unit_ct(u)
        pltpu.make_async_copy(
            tbl.at[c, :, pl.ds(vb, RCH)], slabs[b], isems[b]
        ).wait()

    def compute(b):
        def line(rr):
            vbase = jnp.broadcast_to(rr * 8, (16,)).astype(jnp.int32)
            gs = [
                plsc.load_gather(slabs[b], [eiota, vbase + k]) for k in range(8)
            ]
            for k in range(8):
                stages[b][pl.ds(rr * 128 + k * 16, 16)] = gs[k]

        plsc.parallel_loop(0, ULINES, unroll=4)(line)

    def drain_out(b):
        pltpu.make_async_copy(
            stages[b], out.at[pl.ds(0, UWORDS)], osems[b]
        ).wait()

    def fire_out(u, b):
        c, vb = unit_ct(u)
        pltpu.async_copy(
            stages[b], out.at[pl.ds(c * FWORDS + vb * D, UWORDS)], osems[b]
        )

    @pl.when(wid < MAIN)
    def _():
        fire_in(wid, 0)

    def pair(j, carry):
        u0 = wid + (2 * j) * NW
        u1 = wid + (2 * j + 1) * NW
        u2 = wid + (2 * j + 2) * NW

        @pl.when(u1 < MAIN)
        def _():
            fire_in(u1, 1)

        @pl.when(u0 < MAIN)
        def _():
            drain_in(u0, 0)

            @pl.when(j > 0)
            def _():
                drain_out(0)

            compute(0)
            fire_out(u0, 0)

        @pl.when(u2 < MAIN)
        def _():
            fire_in(u2, 0)

        @pl.when(u1 < MAIN)
        def _():
            drain_in(u1, 1)

            @pl.when(j > 0)
            def _():
                drain_out(1)

            compute(1)
            fire_out(u1, 1)

        return carry

    lax.fori_loop(0, MPAIRS, pair, 0)
    drain_out(0)
    drain_out(1)

    @pl.when(wid < NCAT)
    def _():
        pltpu.sync_copy(tail.at[pl.ds(wid * TWORDS, TWORDS)], tbuf)
        pltpu.sync_copy(tbuf, out.at[pl.ds(wid * FWORDS + VMAIN * D, TWORDS)])


def _sc_relayout(tbl_t, tail1d):
    mesh = plsc.VectorSubcoreMesh(core_axis_name="c", subcore_axis_name="s")
    kern = pl.kernel(
        _sc_relayout_body,
        out_type=jax.ShapeDtypeStruct((NCAT * V * D,), jnp.float32),
        mesh=mesh,
        scratch_types=[
            pltpu.VMEM((D, RCH), jnp.float32),
            pltpu.VMEM((D, RCH), jnp.float32),
            pltpu.VMEM((UWORDS,), jnp.float32),
            pltpu.VMEM((UWORDS,), jnp.float32),
            pltpu.VMEM((TWORDS,), jnp.float32),
            pltpu.SemaphoreType.DMA,
            pltpu.SemaphoreType.DMA,
            pltpu.SemaphoreType.DMA,
            pltpu.SemaphoreType.DMA,
        ],
        compiler_params=pltpu.CompilerParams(
            use_tc_tiling_on_sc=True, needs_layout_passes=False
        ),
    )
    return kern(tbl_t, tail1d)


BS = 2048
NB = B // BS


def _mlp_body(xcat, num, w1c, w1n, b1, g1, be1, w2, b2, g2, be2, w3, b3,
              out, h1_s, h2_s, acc1, acc2, m1, m2):
    p = pl.program_id(0)
    i = pl.program_id(1)
    off = pl.multiple_of(i * BS, BS)

    @pl.when(p == 0)
    def _phase0():
        @pl.when(i == 0)
        def _():
            acc1[...] = jnp.zeros_like(acc1)

        h = jnp.dot(xcat[...], w1c[...], preferred_element_type=jnp.float32)
        h = h + jnp.dot(num[...], w1n[...], preferred_element_type=jnp.float32)
        h = h + b1[...]
        h1_s[pl.ds(off, BS), :] = h
        acc1[0:1, :] += jnp.sum(h, axis=0, keepdims=True)
        acc1[1:2, :] += jnp.sum(h * h, axis=0, keepdims=True)

        @pl.when(i == NB - 1)
        def _():
            mean = acc1[0:1, :] * (1.0 / B)
            var = acc1[1:2, :] * (1.0 / B) - mean * mean
            m1[0:1, :] = mean
            m1[1:2, :] = lax.rsqrt(var + EPS)

    @pl.when(p == 1)
    def _phase1():
        @pl.when(i == 0)
        def _():
            acc2[...] = jnp.zeros_like(acc2)

        h = h1_s[pl.ds(off, BS), :]
        h = (h - m1[0:1, :]) * (m1[1:2, :] * g1[...]) + be1[...]
        h = jnp.maximum(h, 0.0)
        h2 = jnp.dot(h, w2[...], preferred_element_type=jnp.float32) + b2[...]
        h2_s[pl.ds(off, BS), :] = h2
        acc2[0:1, :] += jnp.sum(h2, axis=0, keepdims=True)
        acc2[1:2, :] += jnp.sum(h2 * h2, axis=0, keepdims=True)

        @pl.when(i == NB - 1)
        def _():
            mean = acc2[0:1, :] * (1.0 / B)
            var = acc2[1:2, :] * (1.0 / B) - mean * mean
            m2[0:1, :] = mean
            m2[1:2, :] = lax.rsqrt(var + EPS)

    @pl.when(p == 2)
    def _phase2():
        h2 = h2_s[pl.ds(off, BS), :]
        h2 = (h2 - m2[0:1, :]) * (m2[1:2, :] * g2[...]) + be2[...]
        h2 = jnp.maximum(h2, 0.0)
        o = jnp.sum(h2 * w3[...], axis=1, keepdims=True) + b3[...]
        out[...] = o


def _mlp(xcat_2d, num, w1c_t, w1n_t, b1, g1, be1, w2_t, b2, g2, be2, w3, b3):
    grid = (3, NB)

    def xmap(p, i):
        return (jnp.where(p == 0, i, 0), 0)

    const = lambda p, i: (0, 0)
    return pl.pallas_call(
        _mlp_body,
        grid=grid,
        in_specs=[
            pl.BlockSpec((BS, NCAT * D), xmap),
            pl.BlockSpec((BS, NUM), xmap),
            pl.BlockSpec((NCAT * D, H1), const),
            pl.BlockSpec((NUM, H1), const),
            pl.BlockSpec((1, H1), const),
            pl.BlockSpec((1, H1), const),
            pl.BlockSpec((1, H1), const),
            pl.BlockSpec((H1, H2), const),
            pl.BlockSpec((1, H2), const),
            pl.BlockSpec((1, H2), const),
            pl.BlockSpec((1, H2), const),
            pl.BlockSpec((1, H2), const),
            pl.BlockSpec((1, 1), const),
        ],
        out_specs=pl.BlockSpec((BS, 1), lambda p, i: (i, 0)),
        out_shape=jax.ShapeDtypeStruct((B, 1), jnp.float32),
        scratch_shapes=[
            pltpu.VMEM((B, H1), jnp.float32),
            pltpu.VMEM((B, H2), jnp.float32),
            pltpu.VMEM((2, H1), jnp.float32),
            pltpu.VMEM((2, H2), jnp.float32),
            pltpu.VMEM((2, H1), jnp.float32),
            pltpu.VMEM((2, H2), jnp.float32),
        ],
        compiler_params=pltpu.CompilerParams(
            dimension_semantics=("arbitrary", "arbitrary"),
        ),
    )(xcat_2d, num, w1c_t, w1n_t, b1, g1, be1, w2_t, b2, g2, be2, w3, b3)


def kernel(cat, num, emb_tables, W1, b1, g1, beta1, W2, b2, g2, beta2, W3, b3):
    flat_idx = cat.reshape(R)
    tbl_t = jnp.transpose(emb_tables, (0, 2, 1))
    tail1d = emb_tables[:, VMAIN:, :].reshape(NCAT * TWORDS)
    table = _sc_relayout(tbl_t, tail1d).reshape(NCAT * V, D)

    gathered = _sc_gather(table, flat_idx)
    xcat = gathered.reshape(B, NCAT * D)

    w1t = W1.T
    out = _mlp(
        xcat, num,
        w1t[: NCAT * D, :], w1t[NCAT * D :, :],
        b1.reshape(1, H1), g1.reshape(1, H1), beta1.reshape(1, H1),
        W2.T, b2.reshape(1, H2), g2.reshape(1, H2), beta2.reshape(1, H2),
        W3,
        b3.reshape(1, 1),
    )
    return out


# speedup vs baseline: 1.3046x; 1.0006x over previous
import jax
import jax.numpy as jnp
from jax import lax
from jax.experimental import pallas as pl
from jax.experimental.pallas import tpu as pltpu
from jax.experimental.pallas import tpu_sc as plsc

B = 16384
NCAT = 26
V = 100000
D = 16
NUM = 13
H1 = 128
H2 = 64
EPS = 1e-5

NC = 2
NS = 16
NW = NC * NS
R = B * NCAT
PER_W = R // NW
K = 3328
NCH = PER_W // K
DEPTH = 2


def _sc_gather_body(table, idxs, out, idx_v, *rest):
    rows = rest[:DEPTH]
    gsem = rest[DEPTH : 2 * DEPTH]
    osem = rest[2 * DEPTH : 3 * DEPTH]
    wid = lax.axis_index("s") * NC + lax.axis_index("c")
    base = wid * PER_W

    pltpu.sync_copy(idxs.at[pl.ds(base, PER_W)], idx_v)

    eiota = lax.iota(jnp.int32, 16)

    def fixidx(j):
        phase = (j * 16) % NCAT
        offs = ((eiota + phase) % NCAT) * V
        idx_v[pl.ds(j * 16, 16)] = idx_v[pl.ds(j * 16, 16)] + offs

    plsc.parallel_loop(0, PER_W // 16, unroll=4)(fixidx)

    def fire(j, b):
        return pltpu.async_copy(
            table.at[idx_v.at[pl.ds(j * K, K)]], rows[b], gsem[b]
        )

    g = {}
    o = {}
    for j in range(min(DEPTH, NCH)):
        g[j] = fire(j, j)
    for j in range(NCH):
        b = j % DEPTH
        g[j].wait()
        o[j] = pltpu.async_copy(rows[b], out.at[pl.ds(base + j * K, K)], osem[b])
        n = j + DEPTH
        if n < NCH:
            o[j].wait()
            g[n] = fire(n, b)
    for j in range(max(NCH - DEPTH, 0), NCH):
        o[j].wait()


def _sc_gather(table, flat_idx):
    mesh = plsc.VectorSubcoreMesh(core_axis_name="c", subcore_axis_name="s")
    kern = pl.kernel(
        _sc_gather_body,
        out_type=jax.ShapeDtypeStruct((R, D), jnp.float32),
        mesh=mesh,
        scratch_types=[
            pltpu.VMEM((PER_W,), jnp.int32),
            *[pltpu.VMEM((K, D), jnp.float32) for _ in range(DEPTH)],
            *[pltpu.SemaphoreType.DMA for _ in range(2 * DEPTH)],
        ],
        compiler_params=pltpu.CompilerParams(use_tc_tiling_on_sc=False),
    )
    return kern(table, flat_idx)


VMAIN = (V // 128) * 128
VTAIL = V - VMAIN
RCH = 1408
NCHF = VMAIN // RCH
UPF = NCHF + 1
UNITS = NCAT * UPF
ULINES = RCH * D // 128
UWORDS = RCH * D
FWORDS = V * D
TWORDS = VTAIL * D


NQ = RCH // 128
MAIN = NCAT * NCHF
MPAIRS = ((MAIN + NW - 1) // NW + 1) // 2


def _sc_relayout_body(tbl, tail, out, slab_a, slab_b, stage_a, stage_b, tbuf,
                      isem_a, isem_b, osem_a, osem_b):
    wid = lax.axis_index("s") * NC + lax.axis_index("c")
    eiota = lax.iota(jnp.int32, 16)
    slabs = (slab_a, slab_b)
    stages = (stage_a, stage_b)
    isems = (isem_a, isem_b)
    osems = (osem_a, osem_b)

    def unit_ct(u):
        t = u % NCHF
        return u // NCHF, pl.multiple_of(t * RCH, 128)

    def fire_in(u, b):
        c, vb = unit_ct(u)
        pltpu.async_copy(tbl.at[c, :, pl.ds(vb, RCH)], slabs[b], isems[b])

    def drain_in(u, b):
        c, vb = unit_ct(u)
        pltpu.make_async_copy(
            tbl.at[c, :, pl.ds(vb, RCH)], slabs[b], isems[b]
        ).wait()

    def compute(b):
        def line(rr):
            vbase = jnp.broadcast_to(rr * 8, (16,)).astype(jnp.int32)
            gs = [
                plsc.load_gather(slabs[b], [eiota, vbase + k]) for k in range(8)
            ]
            for k in range(8):
                stages[b][pl.ds(rr * 128 + k * 16, 16)] = gs[k]

        plsc.parallel_loop(0, ULINES, unroll=4)(line)

    def drain_out(b):
        pltpu.make_async_copy(
            stages[b], out.at[pl.ds(0, UWORDS)], osems[b]
        ).wait()

    def fire_out(u, b):
        c, vb = unit_ct(u)
        pltpu.async_copy(
            stages[b], out.at[pl.ds(c * FWORDS + vb * D, UWORDS)], osems[b]
        )

    @pl.when(wid < MAIN)
    def _():
        fire_in(wid, 0)

    def pair(j, carry):
        u0 = wid + (2 * j) * NW
        u1 = wid + (2 * j + 1) * NW
        u2 = wid + (2 * j + 2) * NW

        @pl.when(u1 < MAIN)
        def _():
            fire_in(u1, 1)

        @pl.when(u0 < MAIN)
        def _():
            drain_in(u0, 0)

            @pl.when(j > 0)
            def _():
                drain_out(0)

            compute(0)
            fire_out(u0, 0)

        @pl.when(u2 < MAIN)
        def _():
            fire_in(u2, 0)

        @pl.when(u1 < MAIN)
        def _():
            drain_in(u1, 1)

            @pl.when(j > 0)
            def _():
                drain_out(1)

            compute(1)
            fire_out(u1, 1)

        return carry

    lax.fori_loop(0, MPAIRS, pair, 0)
    drain_out(0)
    drain_out(1)

    @pl.when(wid < NCAT)
    def _():
        pltpu.sync_copy(tail.at[pl.ds(wid * TWORDS, TWORDS)], tbuf)
        pltpu.sync_copy(tbuf, out.at[pl.ds(wid * FWORDS + VMAIN * D, TWORDS)])


def _sc_relayout(tbl_t, tail1d):
    mesh = plsc.VectorSubcoreMesh(core_axis_name="c", subcore_axis_name="s")
    kern = pl.kernel(
        _sc_relayout_body,
        out_type=jax.ShapeDtypeStruct((NCAT * V * D,), jnp.float32),
        mesh=mesh,
        scratch_types=[
            pltpu.VMEM((D, RCH), jnp.float32),
            pltpu.VMEM((D, RCH), jnp.float32),
            pltpu.VMEM((UWORDS,), jnp.float32),
            pltpu.VMEM((UWORDS,), jnp.float32),
            pltpu.VMEM((TWORDS,), jnp.float32),
            pltpu.SemaphoreType.DMA,
            pltpu.SemaphoreType.DMA,
            pltpu.SemaphoreType.DMA,
            pltpu.SemaphoreType.DMA,
        ],
        compiler_params=pltpu.CompilerParams(
            use_tc_tiling_on_sc=True, needs_layout_passes=False
        ),
    )
    return kern(tbl_t, tail1d)


BS = 2048
NB = B // BS


def _mlp_body(xcat, num, w1c, w1n, b1, g1, be1, w2, b2, g2, be2, w3, b3,
              out, h1_s, h2_s, acc1, acc2, m1, m2):
    p = pl.program_id(0)
    i = pl.program_id(1)
    off = pl.multiple_of(i * BS, BS)

    @pl.when(p == 0)
    def _phase0():
        @pl.when(i == 0)
        def _():
            acc1[...] = jnp.zeros_like(acc1)

        h = jnp.dot(xcat[...], w1c[...], preferred_element_type=jnp.float32)
        h = h + jnp.dot(num[...], w1n[...], preferred_element_type=jnp.float32)
        h = h + b1[...]
        h1_s[pl.ds(off, BS), :] = h
        acc1[0:1, :] += jnp.sum(h, axis=0, keepdims=True)
        acc1[1:2, :] += jnp.sum(h * h, axis=0, keepdims=True)

        @pl.when(i == NB - 1)
        def _():
            mean = acc1[0:1, :] * (1.0 / B)
            var = acc1[1:2, :] * (1.0 / B) - mean * mean
            m1[0:1, :] = mean
            m1[1:2, :] = lax.rsqrt(var + EPS)

    @pl.when(p == 1)
    def _phase1():
        @pl.when(i == 0)
        def _():
            acc2[...] = jnp.zeros_like(acc2)

        h = h1_s[pl.ds(off, BS), :]
        h = (h - m1[0:1, :]) * (m1[1:2, :] * g1[...]) + be1[...]
        h = jnp.maximum(h, 0.0)
        h2 = jnp.dot(h, w2[...], preferred_element_type=jnp.float32) + b2[...]
        h2_s[pl.ds(off, BS), :] = h2
        acc2[0:1, :] += jnp.sum(h2, axis=0, keepdims=True)
        acc2[1:2, :] += jnp.sum(h2 * h2, axis=0, keepdims=True)

        @pl.when(i == NB - 1)
        def _():
            mean = acc2[0:1, :] * (1.0 / B)
            var = acc2[1:2, :] * (1.0 / B) - mean * mean
            m2[0:1, :] = mean
            m2[1:2, :] = lax.rsqrt(var + EPS)

    @pl.when(p == 2)
    def _phase2():
        h2 = h2_s[pl.ds(off, BS), :]
        h2 = (h2 - m2[0:1, :]) * (m2[1:2, :] * g2[...]) + be2[...]
        h2 = jnp.maximum(h2, 0.0)
        o = jnp.sum(h2 * w3[...], axis=1, keepdims=True) + b3[...]
        out[...] = o


def _mlp(xcat_2d, num, w1c_t, w1n_t, b1, g1, be1, w2_t, b2, g2, be2, w3, b3):
    grid = (3, NB)

    def xmap(p, i):
        return (jnp.where(p == 0, i, 0), 0)

    const = lambda p, i: (0, 0)
    return pl.pallas_call(
        _mlp_body,
        grid=grid,
        in_specs=[
            pl.BlockSpec((BS, NCAT * D), xmap),
            pl.BlockSpec((BS, NUM), xmap),
            pl.BlockSpec((NCAT * D, H1), const),
            pl.BlockSpec((NUM, H1), const),
            pl.BlockSpec((1, H1), const),
            pl.BlockSpec((1, H1), const),
            pl.BlockSpec((1, H1), const),
            pl.BlockSpec((H1, H2), const),
            pl.BlockSpec((1, H2), const),
            pl.BlockSpec((1, H2), const),
            pl.BlockSpec((1, H2), const),
            pl.BlockSpec((1, H2), const),
            pl.BlockSpec((1, 1), const),
        ],
        out_specs=pl.BlockSpec((BS, 1), lambda p, i: (i, 0)),
        out_shape=jax.ShapeDtypeStruct((B, 1), jnp.float32),
        scratch_shapes=[
            pltpu.VMEM((B, H1), jnp.float32),
            pltpu.VMEM((B, H2), jnp.float32),
            pltpu.VMEM((2, H1), jnp.float32),
            pltpu.VMEM((2, H2), jnp.float32),
            pltpu.VMEM((2, H1), jnp.float32),
            pltpu.VMEM((2, H2), jnp.float32),
        ],
        compiler_params=pltpu.CompilerParams(
            dimension_semantics=("arbitrary", "arbitrary"),
        ),
    )(xcat_2d, num, w1c_t, w1n_t, b1, g1, be1, w2_t, b2, g2, be2, w3, b3)


def kernel(cat, num, emb_tables, W1, b1, g1, beta1, W2, b2, g2, beta2, W3, b3):
    flat_idx = cat.reshape(R)
    tbl_t = jnp.transpose(emb_tables, (0, 2, 1))
    tail1d = emb_tables[:, VMAIN:, :].reshape(NCAT * TWORDS)
    table = _sc_relayout(tbl_t, tail1d).reshape(NCAT * V, D)

    gathered = _sc_gather(table, flat_idx)
    xcat = gathered.reshape(B, NCAT * D)

    w1t = W1.T
    out = _mlp(
        xcat, num,
        w1t[: NCAT * D, :], w1t[NCAT * D :, :],
        b1.reshape(1, H1), g1.reshape(1, H1), beta1.reshape(1, H1),
        W2.T, b2.reshape(1, H2), g2.reshape(1, H2), beta2.reshape(1, H2),
        W3,
        b3.reshape(1, 1),
    )
    return out
